# Initial kernel scaffold; baseline (speedup 1.0000x reference)
import functools

import jax
import jax.numpy as jnp
from jax import lax
from jax.experimental import pallas as pl
from jax.experimental.pallas import tpu as pltpu
from jax.experimental.pallas import tpu_sc as plsc

B = 4096
S = 199
D = 64
ROWS = B * S
NC = 2
NS = 16
NW = NC * NS
RPW = ROWS // NW
CH = 128
STEPS = RPW // CH
PD = S * D


def _add_pos(rows_ref, pos_ref, o):
    def body(r, _):
        pbase = (o + r) * D
        for c in range(D // 16):
            rows_ref[r, pl.ds(c * 16, 16)] = (
                rows_ref[r, pl.ds(c * 16, 16)]
                + pos_ref[pl.ds(pbase + c * 16, 16)]
            )
        return 0

    lax.fori_loop(0, CH, body, 0, unroll=2)


def _sc_body(idx_hbm, table_hbm, pos_hbm, out_hbm,
             idx0, idx1, rows0, rows1, pos_v,
             g0, g1, s0, s1, i0, i1):
    wid = lax.axis_index("s") * NC + lax.axis_index("c")
    base = wid * RPW

    idx_b = (idx0, idx1)
    rows_b = (rows0, rows1)
    g_sem = (g0, g1)
    s_sem = (s0, s1)
    i_sem = (i0, i1)

    def idx_start(t, p):
        pltpu.async_copy(idx_hbm.at[pl.ds(base + t * CH, CH)], idx_b[p],
                         i_sem[p])

    def gather_start(p):
        pltpu.async_copy(table_hbm.at[idx_b[p]], rows_b[p], g_sem[p])

    def scatter_start(t, p):
        pltpu.async_copy(rows_b[p], out_hbm.at[pl.ds(base + t * CH, CH)],
                         s_sem[p])

    def wait(sem, ref):
        pltpu.make_async_copy(ref, ref, sem).wait()

    def compute(t, p):
        o = lax.rem(base + t * CH, S)
        _add_pos(rows_b[p], pos_v, o)

    idx_start(0, 0)
    pltpu.sync_copy(pos_hbm.at[pl.ds(0, PD)], pos_v.at[pl.ds(0, PD)])
    pltpu.sync_copy(pos_hbm.at[pl.ds(0, PD)], pos_v.at[pl.ds(PD, PD)])

    wait(i_sem[0], idx_b[0])
    gather_start(0)
    idx_start(1, 1)
    wait(g_sem[0], rows_b[0])
    idx_start(2, 0)
    wait(i_sem[1], idx_b[1])
    gather_start(1)
    compute(0, 0)
    scatter_start(0, 0)

    def step(t, p):
        wait(g_sem[p], rows_b[p])
        idx_start(t + 2, p)
        wait(s_sem[1 - p], rows_b[1 - p])
        wait(i_sem[1 - p], idx_b[1 - p])
        gather_start(1 - p)
        compute(t, p)
        scatter_start(t, p)

    def pair(i, _):
        t = 1 + 2 * i
        step(t, 1)
        step(t + 1, 0)
        return 0

    lax.fori_loop(0, (STEPS - 3) // 2, pair, 0)

    t = STEPS - 2
    wait(g_sem[1], rows_b[1])
    wait(s_sem[0], rows_b[0])
    wait(i_sem[0], idx_b[0])
    gather_start(0)
    compute(t, 1)
    scatter_start(t, 1)

    t = STEPS - 1
    wait(g_sem[0], rows_b[0])
    wait(s_sem[1], rows_b[1])
    compute(t, 0)
    scatter_start(t, 0)
    wait(s_sem[0], rows_b[0])


@jax.jit
def _run(idx_flat, table, pos_flat):
    mesh = plsc.VectorSubcoreMesh(core_axis_name="c", subcore_axis_name="s")
    f = pl.kernel(
        _sc_body,
        out_type=jax.ShapeDtypeStruct((ROWS, D), jnp.float32),
        mesh=mesh,
        scratch_types=[
            pltpu.VMEM((CH,), jnp.int32),
            pltpu.VMEM((CH,), jnp.int32),
            pltpu.VMEM((CH, D), jnp.float32),
            pltpu.VMEM((CH, D), jnp.float32),
            pltpu.VMEM((2 * PD,), jnp.float32),
            pltpu.SemaphoreType.DMA,
            pltpu.SemaphoreType.DMA,
            pltpu.SemaphoreType.DMA,
            pltpu.SemaphoreType.DMA,
            pltpu.SemaphoreType.DMA,
            pltpu.SemaphoreType.DMA,
        ],
    )
    return f(idx_flat, table, pos_flat)


def kernel(input_e, exercise_table, position_table):
    idx_flat = input_e.reshape(ROWS).astype(jnp.int32)
    pos_flat = position_table[:S].reshape(PD)
    out = _run(idx_flat, exercise_table, pos_flat)
    return out.reshape(B, S, D)


# speedup vs baseline: 3.2953x; 3.2953x over previous
import jax
import jax.numpy as jnp
from jax import lax
from jax.experimental import pallas as pl
from jax.experimental.pallas import tpu as pltpu
from jax.experimental.pallas import tpu_sc as plsc

B = 4096
S = 199
D = 64
ROWS = B * S
NC = 2
NS = 16
NW = NC * NS
RPW = ROWS // NW
CH = 128
STEPS = RPW // CH
PD = S * D


def _add_pos(rows_ref, pos_ref, o):
    def body(r, _):
        pbase = (o + r) * D
        for c in range(D // 16):
            rows_ref[r, pl.ds(c * 16, 16)] = (
                rows_ref[r, pl.ds(c * 16, 16)]
                + pos_ref[pl.ds(pbase + c * 16, 16)]
            )
        return 0

    lax.fori_loop(0, CH, body, 0, unroll=2)


def _sc_body(idx_hbm, table_hbm, pos_hbm, out_hbm,
             idx0, idx1, rows0, rows1, pos_v,
             g0, g1, s0, s1, i0, i1):
    wid = lax.axis_index("s") * NC + lax.axis_index("c")
    base = wid * RPW

    idx_b = (idx0, idx1)
    rows_b = (rows0, rows1)
    g_sem = (g0, g1)
    s_sem = (s0, s1)
    i_sem = (i0, i1)

    def idx_start(t, p):
        pltpu.async_copy(idx_hbm.at[pl.ds(base + t * CH, CH)], idx_b[p],
                         i_sem[p])

    def idx_wait(t, p):
        pltpu.make_async_copy(idx_hbm.at[pl.ds(base + t * CH, CH)],
                              idx_b[p], i_sem[p]).wait()

    def gather_start(p):
        pltpu.async_copy(table_hbm.at[idx_b[p]], rows_b[p], g_sem[p])

    def gather_wait(p):
        pltpu.make_async_copy(table_hbm.at[idx_b[p]], rows_b[p],
                              g_sem[p]).wait()

    def scatter_start(t, p):
        pltpu.async_copy(rows_b[p], out_hbm.at[pl.ds(base + t * CH, CH)],
                         s_sem[p])

    def scatter_wait(t, p):
        pltpu.make_async_copy(rows_b[p], out_hbm.at[pl.ds(base + t * CH, CH)],
                              s_sem[p]).wait()

    def compute(t, p):
        o = lax.rem(base + t * CH, S)
        _add_pos(rows_b[p], pos_v, o)

    idx_start(0, 0)
    pltpu.sync_copy(pos_hbm.at[pl.ds(0, PD)], pos_v.at[pl.ds(0, PD)])
    pltpu.sync_copy(pos_hbm.at[pl.ds(0, PD)], pos_v.at[pl.ds(PD, PD)])

    idx_wait(0, 0)
    gather_start(0)
    idx_start(1, 1)
    gather_wait(0)
    idx_start(2, 0)
    idx_wait(1, 1)
    gather_start(1)
    compute(0, 0)
    scatter_start(0, 0)

    def step(t, p):
        gather_wait(p)
        idx_start(t + 2, p)
        scatter_wait(t - 1, 1 - p)
        idx_wait(t + 1, 1 - p)
        gather_start(1 - p)
        compute(t, p)
        scatter_start(t, p)

    def pair(i, _):
        t = 1 + 2 * i
        step(t, 1)
        step(t + 1, 0)
        return 0

    lax.fori_loop(0, (STEPS - 3) // 2, pair, 0)

    t = STEPS - 2
    gather_wait(1)
    scatter_wait(t - 1, 0)
    idx_wait(t + 1, 0)
    gather_start(0)
    compute(t, 1)
    scatter_start(t, 1)

    t = STEPS - 1
    gather_wait(0)
    scatter_wait(t - 1, 1)
    compute(t, 0)
    scatter_start(t, 0)
    scatter_wait(t, 0)


@jax.jit
def _run(idx_flat, table, pos_flat):
    mesh = plsc.VectorSubcoreMesh(core_axis_name="c", subcore_axis_name="s")
    f = pl.kernel(
        _sc_body,
        out_type=jax.ShapeDtypeStruct((ROWS, D), jnp.float32),
        mesh=mesh,
        compiler_params=pltpu.CompilerParams(use_tc_tiling_on_sc=False),
        scratch_types=[
            pltpu.VMEM((CH,), jnp.int32),
            pltpu.VMEM((CH,), jnp.int32),
            pltpu.VMEM((CH, D), jnp.float32),
            pltpu.VMEM((CH, D), jnp.float32),
            pltpu.VMEM((2 * PD,), jnp.float32),
            pltpu.SemaphoreType.DMA,
            pltpu.SemaphoreType.DMA,
            pltpu.SemaphoreType.DMA,
            pltpu.SemaphoreType.DMA,
            pltpu.SemaphoreType.DMA,
            pltpu.SemaphoreType.DMA,
        ],
    )
    return f(idx_flat, table, pos_flat)


def kernel(input_e, exercise_table, position_table):
    idx_flat = input_e.reshape(ROWS).astype(jnp.int32)
    pos_flat = position_table[:S].reshape(PD)
    out = _run(idx_flat, exercise_table, pos_flat)
    return out.reshape(B, S, D)


# speedup vs baseline: 3.4346x; 1.0423x over previous
import jax
import jax.numpy as jnp
from jax import lax
from jax.experimental import pallas as pl
from jax.experimental.pallas import tpu as pltpu
from jax.experimental.pallas import tpu_sc as plsc

B = 4096
S = 199
D = 64
ROWS = B * S
NC = 2
NS = 16
NW = NC * NS
RPW = ROWS // NW
SB = 128
CH = 512
FULL = RPW // CH
TAIL = RPW - FULL * CH
T = FULL + 1
PD = S * D


def _add_pos(rows_ref, pos_ref, o, rbase):
    def body(r, _):
        pbase = (o + r) * D
        for c in range(D // 16):
            rows_ref[rbase + r, pl.ds(c * 16, 16)] = (
                rows_ref[rbase + r, pl.ds(c * 16, 16)]
                + pos_ref[pl.ds(pbase + c * 16, 16)]
            )
        return 0

    lax.fori_loop(0, SB, body, 0, unroll=2)


def _sc_body(idx_hbm, table_hbm, pos_hbm, out_hbm,
             idx0, idx1, rows0, rows1, pos_v,
             g0, g1, s0, s1, i0, i1):
    wid = lax.axis_index("s") * NC + lax.axis_index("c")
    base = wid * RPW

    idx_b = (idx0, idx1)
    rows_b = (rows0, rows1)
    g_sem = (g0, g1)
    s_sem = (s0, s1)
    i_sem = (i0, i1)

    def idx_start(t, p, n):
        pltpu.async_copy(idx_hbm.at[pl.ds(base + t * CH, n)],
                         idx_b[p].at[pl.ds(0, n)], i_sem[p])

    def idx_wait(t, p, n):
        pltpu.make_async_copy(idx_hbm.at[pl.ds(base + t * CH, n)],
                              idx_b[p].at[pl.ds(0, n)], i_sem[p]).wait()

    def gather_start(p, nsb):
        for j in range(nsb):
            pltpu.async_copy(
                table_hbm.at[idx_b[p].at[pl.ds(j * SB, SB)]],
                rows_b[p].at[pl.ds(j * SB, SB)], g_sem[p])

    def gather_wait(p, nsb):
        for j in range(nsb):
            pltpu.make_async_copy(
                table_hbm.at[idx_b[p].at[pl.ds(j * SB, SB)]],
                rows_b[p].at[pl.ds(j * SB, SB)], g_sem[p]).wait()

    def scatter_start(t, p, n):
        pltpu.async_copy(rows_b[p].at[pl.ds(0, n)],
                         out_hbm.at[pl.ds(base + t * CH, n)], s_sem[p])

    def scatter_wait(t, p, n):
        pltpu.make_async_copy(rows_b[p].at[pl.ds(0, n)],
                              out_hbm.at[pl.ds(base + t * CH, n)],
                              s_sem[p]).wait()

    def compute(t, p, nsb):
        flat = base + t * CH
        for j in range(nsb):
            o = lax.rem(flat + j * SB, S)
            _add_pos(rows_b[p], pos_v, o, j * SB)

    idx_start(0, 0, CH)
    pltpu.sync_copy(pos_hbm.at[pl.ds(0, PD)], pos_v.at[pl.ds(0, PD)])
    pltpu.sync_copy(pos_hbm.at[pl.ds(0, PD)], pos_v.at[pl.ds(PD, PD)])

    idx_wait(0, 0, CH)
    gather_start(0, 4)
    idx_start(1, 1, CH)
    gather_wait(0, 4)
    idx_start(2, 0, CH)
    idx_wait(1, 1, CH)
    gather_start(1, 4)
    compute(0, 0, 4)
    scatter_start(0, 0, CH)

    def step_full(t, p):
        gather_wait(p, 4)
        idx_start(t + 2, p, CH)
        scatter_wait(t - 1, 1 - p, CH)
        idx_wait(t + 1, 1 - p, CH)
        gather_start(1 - p, 4)
        compute(t, p, 4)
        scatter_start(t, p, CH)

    def pair(i, _):
        t = 1 + 2 * i
        step_full(t, 1)
        step_full(t + 1, 0)
        return 0

    lax.fori_loop(0, (FULL - 3) // 2, pair, 0)

    t = FULL - 2
    gather_wait(1, 4)
    idx_start(T - 1, 1, TAIL)
    scatter_wait(t - 1, 0, CH)
    idx_wait(t + 1, 0, CH)
    gather_start(0, 4)
    compute(t, 1, 4)
    scatter_start(t, 1, CH)

    t = FULL - 1
    gather_wait(0, 4)
    scatter_wait(t - 1, 1, CH)
    idx_wait(T - 1, 1, TAIL)
    gather_start(1, TAIL // SB)
    compute(t, 0, 4)
    scatter_start(t, 0, CH)

    t = T - 1
    gather_wait(1, TAIL // SB)
    compute(t, 1, TAIL // SB)
    scatter_start(t, 1, TAIL)
    scatter_wait(t - 1, 0, CH)
    scatter_wait(t, 1, TAIL)


@jax.jit
def _run(idx_flat, table, pos_flat):
    mesh = plsc.VectorSubcoreMesh(core_axis_name="c", subcore_axis_name="s")
    f = pl.kernel(
        _sc_body,
        out_type=jax.ShapeDtypeStruct((ROWS, D), jnp.float32),
        mesh=mesh,
        compiler_params=pltpu.CompilerParams(use_tc_tiling_on_sc=False),
        scratch_types=[
            pltpu.VMEM((CH,), jnp.int32),
            pltpu.VMEM((CH,), jnp.int32),
            pltpu.VMEM((CH, D), jnp.float32),
            pltpu.VMEM((CH, D), jnp.float32),
            pltpu.VMEM((2 * PD,), jnp.float32),
            pltpu.SemaphoreType.DMA,
            pltpu.SemaphoreType.DMA,
            pltpu.SemaphoreType.DMA,
            pltpu.SemaphoreType.DMA,
            pltpu.SemaphoreType.DMA,
            pltpu.SemaphoreType.DMA,
        ],
    )
    return f(idx_flat, table, pos_flat)


def kernel(input_e, exercise_table, position_table):
    idx_flat = input_e.reshape(ROWS).astype(jnp.int32)
    pos_flat = position_table[:S].reshape(PD)
    out = _run(idx_flat, exercise_table, pos_flat)
    return out.reshape(B, S, D)


# speedup vs baseline: 3.4888x; 1.0158x over previous
import jax
import jax.numpy as jnp
from jax import lax
from jax.experimental import pallas as pl
from jax.experimental.pallas import tpu as pltpu
from jax.experimental.pallas import tpu_sc as plsc

B = 4096
S = 199
D = 64
NC = 2
NS = 16
NW = NC * NS
T = B // NW
SB = 128


def _add_pos(rows_ref, pos_ref):
    def body(r, _):
        for c in range(D // 16):
            rows_ref[r, pl.ds(c * 16, 16)] = (
                rows_ref[r, pl.ds(c * 16, 16)]
                + pos_ref[r, pl.ds(c * 16, 16)]
            )
        return 0

    lax.fori_loop(0, S, body, 0, unroll=2)


def _sc_body(idx_hbm, table_hbm, pos_hbm, out_hbm,
             idx0, idx1, rows0, rows1, pos_v,
             g0, g1, s0, s1, i0, i1):
    wid = lax.axis_index("s") * NC + lax.axis_index("c")
    base = wid * T

    idx_b = (idx0, idx1)
    rows_b = (rows0, rows1)
    g_sem = (g0, g1)
    s_sem = (s0, s1)
    i_sem = (i0, i1)

    def idx_start(t, p):
        pltpu.async_copy(idx_hbm.at[base + t], idx_b[p], i_sem[p])

    def idx_wait(t, p):
        pltpu.make_async_copy(idx_hbm.at[base + t], idx_b[p],
                              i_sem[p]).wait()

    def gather_start(p):
        pltpu.async_copy(table_hbm.at[idx_b[p].at[pl.ds(0, SB)]],
                         rows_b[p].at[pl.ds(0, SB)], g_sem[p])
        pltpu.async_copy(table_hbm.at[idx_b[p].at[pl.ds(SB, S - SB)]],
                         rows_b[p].at[pl.ds(SB, S - SB)], g_sem[p])

    def gather_wait(p):
        pltpu.make_async_copy(table_hbm.at[idx_b[p].at[pl.ds(0, SB)]],
                              rows_b[p].at[pl.ds(0, SB)], g_sem[p]).wait()
        pltpu.make_async_copy(table_hbm.at[idx_b[p].at[pl.ds(SB, S - SB)]],
                              rows_b[p].at[pl.ds(SB, S - SB)],
                              g_sem[p]).wait()

    def scatter_start(t, p):
        pltpu.async_copy(rows_b[p], out_hbm.at[base + t], s_sem[p])

    def scatter_wait(t, p):
        pltpu.make_async_copy(rows_b[p], out_hbm.at[base + t],
                              s_sem[p]).wait()

    idx_start(0, 0)
    pltpu.sync_copy(pos_hbm.at[pl.ds(0, S)], pos_v)

    idx_wait(0, 0)
    gather_start(0)
    idx_start(1, 1)
    gather_wait(0)
    idx_start(2, 0)
    idx_wait(1, 1)
    gather_start(1)
    _add_pos(rows_b[0], pos_v)
    scatter_start(0, 0)

    def step(t, p):
        gather_wait(p)
        idx_start(t + 2, p)
        scatter_wait(t - 1, 1 - p)
        idx_wait(t + 1, 1 - p)
        gather_start(1 - p)
        _add_pos(rows_b[p], pos_v)
        scatter_start(t, p)

    def pair(i, _):
        t = 1 + 2 * i
        step(t, 1)
        step(t + 1, 0)
        return 0

    lax.fori_loop(0, (T - 2) // 2 - 1, pair, 0)

    step(T - 3, 1)

    t = T - 2
    gather_wait(0)
    scatter_wait(t - 1, 1)
    idx_wait(t + 1, 1)
    gather_start(1)
    _add_pos(rows_b[0], pos_v)
    scatter_start(t, 0)

    t = T - 1
    gather_wait(1)
    scatter_wait(t - 1, 0)
    _add_pos(rows_b[1], pos_v)
    scatter_start(t, 1)
    scatter_wait(t, 1)


@jax.jit
def _run(input_e, table, position_table):
    mesh = plsc.VectorSubcoreMesh(core_axis_name="c", subcore_axis_name="s")
    f = pl.kernel(
        _sc_body,
        out_type=jax.ShapeDtypeStruct((B, S, D), jnp.float32),
        mesh=mesh,
        compiler_params=pltpu.CompilerParams(use_tc_tiling_on_sc=False),
        scratch_types=[
            pltpu.VMEM((S,), jnp.int32),
            pltpu.VMEM((S,), jnp.int32),
            pltpu.VMEM((S, D), jnp.float32),
            pltpu.VMEM((S, D), jnp.float32),
            pltpu.VMEM((S, D), jnp.float32),
            pltpu.SemaphoreType.DMA,
            pltpu.SemaphoreType.DMA,
            pltpu.SemaphoreType.DMA,
            pltpu.SemaphoreType.DMA,
            pltpu.SemaphoreType.DMA,
            pltpu.SemaphoreType.DMA,
        ],
    )
    return f(input_e, table, position_table)


def kernel(input_e, exercise_table, position_table):
    return _run(input_e.astype(jnp.int32), exercise_table, position_table)
